# baseline (speedup 1.0000x reference)
import functools
import math

import jax
import jax.numpy as jnp
from jax import lax
from jax.experimental import pallas as pl
from jax.experimental.pallas import tpu as pltpu
from jax.experimental.pallas import tpu_sc as plsc

V = 100000
B = 1024
S_VAL = 0.1 / (V - 2)
C_VAL = 0.9
K_CONST = (V - 2) * S_VAL * math.log(S_VAL) + C_VAL * math.log(C_VAL)

RB = 32
NJ = B // RB

NC = 2
NS = 16
NW = NC * NS
BPW = B // NW
LANES = 16


def _tc_body(t_ref, x_ref, rs_ref, tv_ref):
    x = x_ref[...]
    t = t_ref[0, 0, :]
    cols = lax.broadcasted_iota(jnp.int32, (RB, V), 1)
    tv = jnp.sum(jnp.where(cols == t[:, None], x, 0.0), axis=1)
    rs = jnp.sum(x, axis=1) - x[:, 0]
    rs_ref[...] = rs[None, None, :]
    tv_ref[...] = tv[None, None, :]


def _tc_stream(output, target3):
    return pl.pallas_call(
        _tc_body,
        grid=(NJ,),
        in_specs=[pl.BlockSpec((1, 1, RB), lambda j: (j, 0, 0)),
                  pl.BlockSpec((RB, V), lambda j: (j, 0))],
        out_specs=[pl.BlockSpec((1, 1, RB), lambda j: (j, 0, 0)),
                   pl.BlockSpec((1, 1, RB), lambda j: (j, 0, 0))],
        out_shape=[jax.ShapeDtypeStruct((NJ, 1, RB), jnp.float32),
                   jax.ShapeDtypeStruct((NJ, 1, RB), jnp.float32)],
        compiler_params=pltpu.CompilerParams(
            dimension_semantics=("arbitrary",),
        ),
    )(target3, output)


def _sc_body(tgt_hbm, rsum_hbm, tval_hbm, out_hbm,
             tgt_v, rs_v, tv_v, acc_v, all_v, shared, sem):
    c = lax.axis_index("c")
    s = lax.axis_index("s")
    wid = s * NC + c
    base = wid * BPW

    pltpu.sync_copy(tgt_hbm.at[pl.ds(base, BPW)], tgt_v)
    pltpu.sync_copy(rsum_hbm.at[pl.ds(base, BPW)], rs_v)
    pltpu.sync_copy(tval_hbm.at[pl.ds(base, BPW)], tv_v)

    acc = jnp.zeros((LANES,), jnp.float32)
    for k in range(BPW // LANES):
        tvec = tgt_v[pl.ds(k * LANES, LANES)]
        rvec = rs_v[pl.ds(k * LANES, LANES)]
        tval = tv_v[pl.ds(k * LANES, LANES)]
        maskf = jnp.where(tvec != 0, 1.0, 0.0)
        acc = acc + maskf * (K_CONST - S_VAL * rvec + (S_VAL - C_VAL) * tval)
    acc_v[...] = acc

    pltpu.sync_copy(acc_v, shared.at[s])
    plsc.subcore_barrier()

    @pl.when(s == 0)
    def _reduce():
        pltpu.sync_copy(shared, all_v)
        tot = jnp.zeros((LANES,), jnp.float32)
        for i in range(NS):
            tot = tot + all_v[i]
        acc_v[...] = tot
        pltpu.sync_copy(acc_v, out_hbm.at[c])


@functools.lru_cache(maxsize=1)
def _sc_combine():
    return functools.partial(
        pl.kernel,
        mesh=plsc.VectorSubcoreMesh(core_axis_name="c", subcore_axis_name="s"),
        out_type=jax.ShapeDtypeStruct((NC, LANES), jnp.float32),
        compiler_params=pltpu.CompilerParams(use_tc_tiling_on_sc=False),
        scratch_types=[
            pltpu.VMEM((BPW,), jnp.int32),
            pltpu.VMEM((BPW,), jnp.float32),
            pltpu.VMEM((BPW,), jnp.float32),
            pltpu.VMEM((LANES,), jnp.float32),
            pltpu.VMEM((NS, LANES), jnp.float32),
            pltpu.VMEM_SHARED((NS, LANES), jnp.float32),
            pltpu.SemaphoreType.DMA,
        ],
    )(_sc_body)


def _sc_probe_body(out2d_hbm, dst_hbm, buf, acc_v, sem):
    c = lax.axis_index("c")
    s = lax.axis_index("s")
    wid = s * NC + c
    r0 = 768 + wid * 8
    for chunk in range(24):
        pltpu.sync_copy(
            out2d_hbm.at[pl.ds(r0, 8), pl.ds(chunk * 4096, 4096)], buf)

    @pl.when((s == 0) & (c == 0))
    def _():
        acc_v[...] = jnp.zeros((LANES,), jnp.float32)
        pltpu.sync_copy(acc_v, dst_hbm.at[0])
        pltpu.sync_copy(acc_v, dst_hbm.at[1])


@functools.lru_cache(maxsize=1)
def _sc_probe():
    return functools.partial(
        pl.kernel,
        mesh=plsc.VectorSubcoreMesh(core_axis_name="c", subcore_axis_name="s"),
        out_type=jax.ShapeDtypeStruct((NC, LANES), jnp.float32),
        compiler_params=pltpu.CompilerParams(use_tc_tiling_on_sc=True),
        scratch_types=[
            pltpu.VMEM((8, 4096), jnp.float32),
            pltpu.VMEM((LANES,), jnp.float32),
            pltpu.SemaphoreType.DMA,
        ],
    )(_sc_probe_body)


def kernel(output, target, one_hot):
    del one_hot
    target3 = target.reshape(NJ, 1, RB)
    rowsums, tvals = _tc_stream(output, target3)
    parts = _sc_combine()(target, rowsums.reshape(B), tvals.reshape(B))
    zeros2 = _sc_probe()(output)
    return jnp.sum(parts) + jnp.sum(zeros2)


# speedup vs baseline: 1.1112x; 1.1112x over previous
import functools
import math

import jax
import jax.numpy as jnp
from jax import lax
from jax.experimental import pallas as pl
from jax.experimental.pallas import tpu as pltpu
from jax.experimental.pallas import tpu_sc as plsc

V = 100000
B = 1024
S_VAL = 0.1 / (V - 2)
C_VAL = 0.9
K_CONST = (V - 2) * S_VAL * math.log(S_VAL) + C_VAL * math.log(C_VAL)

RB = 32
NJ = B // RB

NC = 2
NS = 16
NW = NC * NS
BPW = B // NW
LANES = 16


def _tc_body(t_ref, x_ref, rs_ref, tv_ref):
    x = x_ref[...]
    t = t_ref[0, 0, :]
    cols = lax.broadcasted_iota(jnp.int32, (RB, V), 1)
    tv = jnp.sum(jnp.where(cols == t[:, None], x, 0.0), axis=1)
    rs = jnp.sum(x, axis=1) - x[:, 0]
    rs_ref[...] = rs[None, None, :]
    tv_ref[...] = tv[None, None, :]


def _tc_stream(output, target3):
    return pl.pallas_call(
        _tc_body,
        grid=(NJ,),
        in_specs=[pl.BlockSpec((1, 1, RB), lambda j: (j, 0, 0)),
                  pl.BlockSpec((RB, V), lambda j: (j, 0))],
        out_specs=[pl.BlockSpec((1, 1, RB), lambda j: (j, 0, 0)),
                   pl.BlockSpec((1, 1, RB), lambda j: (j, 0, 0))],
        out_shape=[jax.ShapeDtypeStruct((NJ, 1, RB), jnp.float32),
                   jax.ShapeDtypeStruct((NJ, 1, RB), jnp.float32)],
        compiler_params=pltpu.CompilerParams(
            dimension_semantics=("arbitrary",),
        ),
    )(target3, output)


def _sc_body(tgt_hbm, rsum_hbm, tval_hbm, out_hbm,
             tgt_v, rs_v, tv_v, acc_v, all_v, shared, sem):
    c = lax.axis_index("c")
    s = lax.axis_index("s")
    wid = s * NC + c
    base = wid * BPW

    pltpu.sync_copy(tgt_hbm.at[pl.ds(base, BPW)], tgt_v)
    pltpu.sync_copy(rsum_hbm.at[pl.ds(base, BPW)], rs_v)
    pltpu.sync_copy(tval_hbm.at[pl.ds(base, BPW)], tv_v)

    acc = jnp.zeros((LANES,), jnp.float32)
    for k in range(BPW // LANES):
        tvec = tgt_v[pl.ds(k * LANES, LANES)]
        rvec = rs_v[pl.ds(k * LANES, LANES)]
        tval = tv_v[pl.ds(k * LANES, LANES)]
        maskf = jnp.where(tvec != 0, 1.0, 0.0)
        acc = acc + maskf * (K_CONST - S_VAL * rvec + (S_VAL - C_VAL) * tval)
    acc_v[...] = acc

    pltpu.sync_copy(acc_v, shared.at[s])
    plsc.subcore_barrier()

    @pl.when(s == 0)
    def _reduce():
        pltpu.sync_copy(shared, all_v)
        tot = jnp.zeros((LANES,), jnp.float32)
        for i in range(NS):
            tot = tot + all_v[i]
        acc_v[...] = tot
        pltpu.sync_copy(acc_v, out_hbm.at[c])


@functools.lru_cache(maxsize=1)
def _sc_combine():
    return functools.partial(
        pl.kernel,
        mesh=plsc.VectorSubcoreMesh(core_axis_name="c", subcore_axis_name="s"),
        out_type=jax.ShapeDtypeStruct((NC, LANES), jnp.float32),
        compiler_params=pltpu.CompilerParams(use_tc_tiling_on_sc=False),
        scratch_types=[
            pltpu.VMEM((BPW,), jnp.int32),
            pltpu.VMEM((BPW,), jnp.float32),
            pltpu.VMEM((BPW,), jnp.float32),
            pltpu.VMEM((LANES,), jnp.float32),
            pltpu.VMEM((NS, LANES), jnp.float32),
            pltpu.VMEM_SHARED((NS, LANES), jnp.float32),
            pltpu.SemaphoreType.DMA,
        ],
    )(_sc_body)


def kernel(output, target, one_hot):
    del one_hot
    target3 = target.reshape(NJ, 1, RB)
    rowsums, tvals = _tc_stream(output, target3)
    parts = _sc_combine()(target, rowsums.reshape(B), tvals.reshape(B))
    return jnp.sum(parts)
